# baseline (speedup 1.0000x reference)
import functools

import jax
import jax.numpy as jnp
import numpy as np
from jax import lax
from jax.experimental import pallas as pl
from jax.experimental.pallas import tpu as pltpu
from jax.experimental.pallas import tpu_sc as plsc

_N = 10000
_E = 320000
_D = 128
_NC = 2
_NS = 16
_K = 125
_NW = _NC * _NS
_CHUNKS = _E // (_NW * _K)
_NP = 10240
_NPA = 10112
_RPT = _NPA // _NS

_NBUF = 2


_GRP = 8
_NGRP = _CHUNKS // _GRP


def _sc_agg_body(with_deg, *refs):
    if with_deg:
        (src_r, dst_r, hw_r, z_r, agg_o, deg_o,
         sidx, didx, rows, hist, acc, *sems) = refs
    else:
        (src_r, dst_r, hw_r, z_r, agg_o,
         sidx, didx, rows, acc, *sems) = refs
    gsems, ssems, dsem, xsem = sems[:_NBUF], sems[_NBUF:2 * _NBUF], sems[-2], sems[-1]
    c = lax.axis_index("c")
    s = lax.axis_index("s")
    w = c * _NS + s
    tb = s * _RPT
    pltpu.sync_copy(z_r.at[pl.ds(tb, _RPT)], acc.at[pl.ds(tb, _RPT)])
    if with_deg:
        def zero(i, carry):
            hist[pl.ds(i * 16, 16)] = jnp.zeros((16,), jnp.float32)
            return carry

        lax.fori_loop(0, _NP // 16, zero, 0)
    plsc.subcore_barrier()

    def grp_start(arr_r, ring, sem, g):
        pltpu.async_copy(arr_r.at[pl.ds(w * _CHUNKS + g * _GRP, _GRP)],
                         ring.at[g % 2], sem)

    def grp_wait(arr_r, ring, sem, g):
        pltpu.make_async_copy(arr_r.at[pl.ds(w * _CHUNKS + g * _GRP, _GRP)],
                              ring.at[g % 2], sem).wait()

    def _row_of(ring, j):
        return ring.at[(j // _GRP) % 2, j % _GRP]

    def gather_start(j, b):
        pltpu.async_copy(hw_r.at[_row_of(sidx, j)], rows.at[b], gsems[b])

    def gather_wait(j, b):
        pltpu.make_async_copy(hw_r.at[_row_of(sidx, j)], rows.at[b],
                              gsems[b]).wait()

    def scatter_start(j, b):
        pltpu.async_copy(rows.at[b], acc.at[_row_of(didx, j)], ssems[b],
                         add=True)

    def scatter_wait(j, b):
        pltpu.make_async_copy(rows.at[b], acc.at[_row_of(didx, j)],
                              ssems[b]).wait()

    ones = jnp.ones((16,), jnp.float32)
    tail = lax.iota(jnp.int32, 16) >= (16 - _K % 16)

    def deg_row(j):
        slot = (j // _GRP) % 2
        r = j % _GRP
        for k in range(_K // 16):
            plsc.addupdate_scatter(hist, [didx[slot, r, pl.ds(k * 16, 16)]],
                                   ones)
        plsc.addupdate_scatter(hist, [didx[slot, r, pl.ds(_K - 16, 16)]],
                               ones, mask=tail)

    grp_start(src_r, sidx, xsem, 0)
    grp_wait(src_r, sidx, xsem, 0)
    grp_start(dst_r, didx, dsem, 0)
    grp_wait(dst_r, didx, dsem, 0)
    grp_start(src_r, sidx, xsem, 1)
    grp_start(dst_r, didx, dsem, 1)
    for b in range(_NBUF):
        gather_start(b, b)

    def step(q, carry):
        for b in range(_NBUF):
            j = q * _NBUF + b
            jp = j - 1
            bp = (b - 1) % _NBUF

            @pl.when(jp >= 0)
            def _():
                scatter_wait(jp, bp)
                jn = jp + _NBUF

                @pl.when(jn < _CHUNKS)
                def _():
                    gather_start(jn, bp)

                g = j // _GRP

                @pl.when(j % _GRP == 0)
                def _():
                    grp_wait(dst_r, didx, dsem, g)

                    @pl.when(g + 1 < _NGRP)
                    def _():
                        grp_start(dst_r, didx, dsem, g + 1)
                        grp_start(src_r, sidx, xsem, g + 1)

                @pl.when((j % _GRP == _GRP - 2) & (g + 1 < _NGRP))
                def _():
                    grp_wait(src_r, sidx, xsem, g + 1)

            gather_wait(j, b)
            scatter_start(j, b)
            if with_deg:
                deg_row(j)

        return carry

    lax.fori_loop(0, _CHUNKS // _NBUF, step, 0)
    scatter_wait(_CHUNKS - 1, (_CHUNKS - 1) % _NBUF)
    plsc.subcore_barrier()
    pltpu.sync_copy(acc.at[pl.ds(tb, _RPT)], agg_o.at[c, pl.ds(tb, _RPT)])
    if with_deg:
        pltpu.sync_copy(hist, deg_o.at[w])


def _mesh():
    return plsc.VectorSubcoreMesh(core_axis_name="c", subcore_axis_name="s",
                                  num_cores=_NC, num_subcores=_NS)


_Z128 = np.zeros((_NPA, _D), np.float32)


def _make_sc_agg(with_deg):
    out_type = [jax.ShapeDtypeStruct((_NC, _NPA, _D), jnp.float32)]
    scratch = [
        pltpu.VMEM((2, _GRP, _K), jnp.int32),
        pltpu.VMEM((2, _GRP, _K), jnp.int32),
        pltpu.VMEM((_NBUF, _K, _D), jnp.float32),
        pltpu.VMEM_SHARED((_NPA, _D), jnp.float32),
    ]
    if with_deg:
        out_type.append(jax.ShapeDtypeStruct((_NW, _NP), jnp.float32))
        scratch.insert(3, pltpu.VMEM((_NP,), jnp.float32))
    return pl.kernel(
        functools.partial(_sc_agg_body, with_deg),
        out_type=tuple(out_type) if with_deg else out_type[0],
        mesh=_mesh(),
        scratch_types=scratch + [pltpu.SemaphoreType.DMA] * (2 * _NBUF + 2),
        compiler_params=pltpu.CompilerParams(needs_layout_passes=False),
    )


def _sc_agg_deg(src2, dst2, hw):
    return _make_sc_agg(True)(src2, dst2, hw, _Z128)


def _sc_agg(src2, dst2, hw):
    return _make_sc_agg(False)(src2, dst2, hw, _Z128)


_BM = 5000


def _tc1_body(x_r, ws_r, wn_r, b_r, hs_o, hw_o):
    x = x_r[...]
    dn = (((1,), (1,)), ((), ()))
    hs_o[...] = lax.dot_general(x, ws_r[...], dn,
                                preferred_element_type=jnp.float32) + b_r[...]
    hw_o[...] = lax.dot_general(x, wn_r[...], dn,
                                preferred_element_type=jnp.float32)


def _tc_inv_body(deg_r, inv_o):
    deg = jnp.sum(deg_r[...], axis=0)[:, None]
    inv_o[...] = 1.0 / jnp.maximum(deg, 1.0)


def _tc2_body(hs_r, agg_r, inv_r, ws_r, wn_r, b_r, h1s_o, h1w_o):
    h1 = hs_r[...] + (agg_r[0] + agg_r[1]) * inv_r[...]
    dn = (((1,), (1,)), ((), ()))
    h1s_o[...] = lax.dot_general(h1, ws_r[...], dn,
                                 preferred_element_type=jnp.float32) + b_r[...]
    h1w_o[...] = lax.dot_general(h1, wn_r[...], dn,
                                 preferred_element_type=jnp.float32)


def _tc3_body(hs_r, agg_r, inv_r, out_o):
    out_o[...] = hs_r[...] + (agg_r[0] + agg_r[1]) * inv_r[...]


_ROWS_SPEC = pl.BlockSpec((_BM, _D), lambda i: (i, 0))
_AGG_SPEC = pl.BlockSpec((_NC, _BM, _D), lambda i: (0, i, 0))
_INV_SPEC = pl.BlockSpec((_BM, 1), lambda i: (i, 0))


def _tc_inv(deg):
    return pl.pallas_call(
        _tc_inv_body,
        grid=(1,),
        in_specs=[pl.BlockSpec((_NW, _NP), lambda i: (0, 0))],
        out_specs=pl.BlockSpec((_NP, 1), lambda i: (0, 0)),
        out_shape=jax.ShapeDtypeStruct((_NP, 1), jnp.float32),
    )(deg)
_W_SPEC = pl.BlockSpec((_D, _D), lambda i: (0, 0))
_B_SPEC = pl.BlockSpec((1, _D), lambda i: (0, 0))


def _tc1(x, ws, wn, b):
    return pl.pallas_call(
        _tc1_body,
        grid=(_N // _BM,),
        in_specs=[_ROWS_SPEC, _W_SPEC, _W_SPEC, _B_SPEC],
        out_specs=[_ROWS_SPEC, _ROWS_SPEC],
        out_shape=[jax.ShapeDtypeStruct((_N, _D), jnp.float32)] * 2,
    )(x, ws, wn, b)


def _tc2(hs, agg, inv, ws, wn, b):
    return pl.pallas_call(
        _tc2_body,
        grid=(_N // _BM,),
        in_specs=[_ROWS_SPEC, _AGG_SPEC, _INV_SPEC, _W_SPEC, _W_SPEC, _B_SPEC],
        out_specs=[_ROWS_SPEC, _ROWS_SPEC],
        out_shape=[jax.ShapeDtypeStruct((_N, _D), jnp.float32)] * 2,
    )(hs, agg, inv, ws, wn, b)


def _tc3(hs, agg, inv):
    return pl.pallas_call(
        _tc3_body,
        grid=(_N // _BM,),
        in_specs=[_ROWS_SPEC, _AGG_SPEC, _INV_SPEC],
        out_specs=_ROWS_SPEC,
        out_shape=jax.ShapeDtypeStruct((_N, _D), jnp.float32),
    )(hs, agg, inv)


def kernel(in_feat, edge_index, W_self_0, W_neigh_0, b_0, W_self_1, W_neigh_1, b_1):
    src2 = edge_index[0].reshape(_E // _K, _K)
    dst2 = edge_index[1].reshape(_E // _K, _K)
    b0 = b_0.reshape(1, _D)
    b1 = b_1.reshape(1, _D)

    hs0, hw0 = _tc1(in_feat, W_self_0, W_neigh_0, b0)
    agg0, deg = _sc_agg_deg(src2, dst2, hw0)
    inv = _tc_inv(deg)
    h1s, h1w = _tc2(hs0, agg0, inv, W_self_1, W_neigh_1, b1)
    agg1 = _sc_agg(src2, dst2, h1w)
    return _tc3(h1s, agg1, inv)


# speedup vs baseline: 1.0190x; 1.0190x over previous
import jax
import jax.numpy as jnp
import numpy as np
from jax import lax
from jax.experimental import pallas as pl
from jax.experimental.pallas import tpu as pltpu
from jax.experimental.pallas import tpu_sc as plsc

_N = 10000
_E = 320000
_D = 128
_NC = 2
_NS = 16
_K = 125
_NW = _NC * _NS
_CHUNKS = _E // (_NW * _K)
_NP = 10240
_NPA = 10112
_RPT = _NPA // _NS

_NBUF = 2


_GRP = 8
_NGRP = _CHUNKS // _GRP


def _sc_agg_body(src_r, dst_r, hw_r, z_r, agg_o, sidx, didx, rows, acc, *sems):
    gsems, ssems, dsem = sems[:_NBUF], sems[_NBUF:2 * _NBUF], sems[2 * _NBUF]
    c = lax.axis_index("c")
    s = lax.axis_index("s")
    w = c * _NS + s
    tb = s * _RPT
    pltpu.sync_copy(z_r.at[pl.ds(tb, _RPT)], acc.at[pl.ds(tb, _RPT)])
    pltpu.sync_copy(src_r.at[pl.ds(w * _CHUNKS, _CHUNKS)], sidx)
    plsc.subcore_barrier()

    def didx_start(g):
        pltpu.async_copy(dst_r.at[pl.ds(w * _CHUNKS + g * _GRP, _GRP)],
                         didx.at[g % 2], dsem)

    def didx_wait(g):
        pltpu.make_async_copy(dst_r.at[pl.ds(w * _CHUNKS + g * _GRP, _GRP)],
                              didx.at[g % 2], dsem).wait()

    def gather_start(j, b):
        pltpu.async_copy(hw_r.at[sidx.at[j]], rows.at[b], gsems[b])

    def gather_wait(j, b):
        pltpu.make_async_copy(hw_r.at[sidx.at[j]], rows.at[b], gsems[b]).wait()

    def _didx_of(j):
        return didx.at[(j // _GRP) % 2, j % _GRP]

    def scatter_start(j, b):
        pltpu.async_copy(rows.at[b], acc.at[_didx_of(j)], ssems[b], add=True)

    def scatter_wait(j, b):
        pltpu.make_async_copy(rows.at[b], acc.at[_didx_of(j)], ssems[b]).wait()

    didx_start(0)
    didx_wait(0)
    didx_start(1)
    for b in range(_NBUF):
        gather_start(b, b)

    def step(q, carry):
        for b in range(_NBUF):
            j = q * _NBUF + b
            jp = j - 1
            bp = (b - 1) % _NBUF

            @pl.when(jp >= 0)
            def _():
                scatter_wait(jp, bp)
                jn = jp + _NBUF

                @pl.when(jn < _CHUNKS)
                def _():
                    gather_start(jn, bp)

                g = j // _GRP

                @pl.when(j % _GRP == 0)
                def _():
                    didx_wait(g)

                    @pl.when(g + 1 < _NGRP)
                    def _():
                        didx_start(g + 1)

            gather_wait(j, b)
            scatter_start(j, b)

        return carry

    lax.fori_loop(0, _CHUNKS // _NBUF, step, 0)
    scatter_wait(_CHUNKS - 1, (_CHUNKS - 1) % _NBUF)
    plsc.subcore_barrier()
    pltpu.sync_copy(acc.at[pl.ds(tb, _RPT)], agg_o.at[c, pl.ds(tb, _RPT)])


_EPW = _E // _NW


def _sc_deg_body(dst_r, deg_o, didx, hist, sem):
    c = lax.axis_index("c")
    s = lax.axis_index("s")
    w = c * _NS + s
    pltpu.sync_copy(dst_r.at[pl.ds(w * _EPW, _EPW)], didx)

    def zero(i, carry):
        hist[pl.ds(i * 16, 16)] = jnp.zeros((16,), jnp.float32)
        return carry

    lax.fori_loop(0, _NP // 16, zero, 0)
    ones = jnp.ones((16,), jnp.float32)

    def step(i, carry):
        plsc.addupdate_scatter(hist, [didx[pl.ds(i * 16, 16)]], ones)
        return carry

    lax.fori_loop(0, _EPW // 16, step, 0)
    pltpu.sync_copy(hist, deg_o.at[w])


def _mesh():
    return plsc.VectorSubcoreMesh(core_axis_name="c", subcore_axis_name="s",
                                  num_cores=_NC, num_subcores=_NS)


_Z128 = np.zeros((_NPA, _D), np.float32)


def _sc_agg(src2, dst2, hw):
    return pl.kernel(
        _sc_agg_body,
        out_type=jax.ShapeDtypeStruct((_NC, _NPA, _D), jnp.float32),
        mesh=_mesh(),
        scratch_types=[
            pltpu.VMEM((_CHUNKS, _K), jnp.int32),
            pltpu.VMEM((2, _GRP, _K), jnp.int32),
            pltpu.VMEM((_NBUF, _K, _D), jnp.float32),
            pltpu.VMEM_SHARED((_NPA, _D), jnp.float32),
        ] + [pltpu.SemaphoreType.DMA] * (2 * _NBUF + 1),
    )(src2, dst2, hw, _Z128)


def _sc_deg(dst1d):
    return pl.kernel(
        _sc_deg_body,
        out_type=jax.ShapeDtypeStruct((_NW, _NP), jnp.float32),
        mesh=_mesh(),
        scratch_types=[
            pltpu.VMEM((_EPW,), jnp.int32),
            pltpu.VMEM((_NP,), jnp.float32),
            pltpu.SemaphoreType.DMA,
        ],
        compiler_params=pltpu.CompilerParams(needs_layout_passes=False),
    )(dst1d)


_BM = 5000


def _tc1_body(x_r, ws_r, wn_r, b_r, hs_o, hw_o):
    x = x_r[...]
    dn = (((1,), (1,)), ((), ()))
    hs_o[...] = lax.dot_general(x, ws_r[...], dn,
                                preferred_element_type=jnp.float32) + b_r[...]
    hw_o[...] = lax.dot_general(x, wn_r[...], dn,
                                preferred_element_type=jnp.float32)


def _tc_inv_body(deg_r, inv_o):
    deg = jnp.sum(deg_r[...], axis=0)[:, None]
    inv_o[...] = 1.0 / jnp.maximum(deg, 1.0)


def _tc2_body(hs_r, agg_r, inv_r, ws_r, wn_r, b_r, h1s_o, h1w_o):
    h1 = hs_r[...] + (agg_r[0] + agg_r[1]) * inv_r[...]
    dn = (((1,), (1,)), ((), ()))
    h1s_o[...] = lax.dot_general(h1, ws_r[...], dn,
                                 preferred_element_type=jnp.float32) + b_r[...]
    h1w_o[...] = lax.dot_general(h1, wn_r[...], dn,
                                 preferred_element_type=jnp.float32)


def _tc3_body(hs_r, agg_r, inv_r, out_o):
    out_o[...] = hs_r[...] + (agg_r[0] + agg_r[1]) * inv_r[...]


_ROWS_SPEC = pl.BlockSpec((_BM, _D), lambda i: (i, 0))
_AGG_SPEC = pl.BlockSpec((_NC, _BM, _D), lambda i: (0, i, 0))
_INV_SPEC = pl.BlockSpec((_BM, 1), lambda i: (i, 0))


def _tc_inv(deg):
    return pl.pallas_call(
        _tc_inv_body,
        grid=(1,),
        in_specs=[pl.BlockSpec((_NW, _NP), lambda i: (0, 0))],
        out_specs=pl.BlockSpec((_NP, 1), lambda i: (0, 0)),
        out_shape=jax.ShapeDtypeStruct((_NP, 1), jnp.float32),
    )(deg)
_W_SPEC = pl.BlockSpec((_D, _D), lambda i: (0, 0))
_B_SPEC = pl.BlockSpec((1, _D), lambda i: (0, 0))


def _tc1(x, ws, wn, b):
    return pl.pallas_call(
        _tc1_body,
        grid=(_N // _BM,),
        in_specs=[_ROWS_SPEC, _W_SPEC, _W_SPEC, _B_SPEC],
        out_specs=[_ROWS_SPEC, _ROWS_SPEC],
        out_shape=[jax.ShapeDtypeStruct((_N, _D), jnp.float32)] * 2,
    )(x, ws, wn, b)


def _tc2(hs, agg, inv, ws, wn, b):
    return pl.pallas_call(
        _tc2_body,
        grid=(_N // _BM,),
        in_specs=[_ROWS_SPEC, _AGG_SPEC, _INV_SPEC, _W_SPEC, _W_SPEC, _B_SPEC],
        out_specs=[_ROWS_SPEC, _ROWS_SPEC],
        out_shape=[jax.ShapeDtypeStruct((_N, _D), jnp.float32)] * 2,
    )(hs, agg, inv, ws, wn, b)


def _tc3(hs, agg, inv):
    return pl.pallas_call(
        _tc3_body,
        grid=(_N // _BM,),
        in_specs=[_ROWS_SPEC, _AGG_SPEC, _INV_SPEC],
        out_specs=_ROWS_SPEC,
        out_shape=jax.ShapeDtypeStruct((_N, _D), jnp.float32),
    )(hs, agg, inv)


def kernel(in_feat, edge_index, W_self_0, W_neigh_0, b_0, W_self_1, W_neigh_1, b_1):
    src2 = edge_index[0].reshape(_E // _K, _K)
    dst2 = edge_index[1].reshape(_E // _K, _K)
    b0 = b_0.reshape(1, _D)
    b1 = b_1.reshape(1, _D)

    inv = _tc_inv(_sc_deg(edge_index[1]))
    hs0, hw0 = _tc1(in_feat, W_self_0, W_neigh_0, b0)
    agg0 = _sc_agg(src2, dst2, hw0)
    h1s, h1w = _tc2(hs0, agg0, inv, W_self_1, W_neigh_1, b1)
    agg1 = _sc_agg(src2, dst2, h1w)
    return _tc3(h1s, agg1, inv)
